# baseline (speedup 1.0000x reference)
import functools

import jax
import jax.numpy as jnp
from jax import lax
from jax.experimental import pallas as pl
from jax.experimental.pallas import tpu as pltpu
from jax.experimental.pallas import tpu_sc as plsc

N_WORD = 128
B = 4096
L = 50
LP = L + 2

NC = 2
NS = 16
NW = NC * NS
BW = B // NW
NBUF = 5


def _sc_embed(table, idx_t):
    mesh = plsc.VectorSubcoreMesh(core_axis_name="c", subcore_axis_name="s")

    @functools.partial(
        pl.kernel,
        mesh=mesh,
        out_type=jax.ShapeDtypeStruct((LP, B, N_WORD), jnp.float32),
        scratch_types=[
            pltpu.VMEM((L, BW), jnp.int32),
            pltpu.VMEM((BW, N_WORD), jnp.float32),
            pltpu.VMEM((BW, N_WORD), jnp.float32),
            pltpu.VMEM((BW, N_WORD), jnp.float32),
            pltpu.VMEM((BW, N_WORD), jnp.float32),
            pltpu.VMEM((BW, N_WORD), jnp.float32),
            pltpu.VMEM((BW, N_WORD), jnp.float32),
            pltpu.SemaphoreType.DMA,
            pltpu.SemaphoreType.DMA,
            pltpu.SemaphoreType.DMA,
            pltpu.SemaphoreType.DMA,
            pltpu.SemaphoreType.DMA,
            pltpu.SemaphoreType.DMA,
            pltpu.SemaphoreType.DMA,
            pltpu.SemaphoreType.DMA,
            pltpu.SemaphoreType.DMA,
            pltpu.SemaphoreType.DMA,
            pltpu.SemaphoreType.DMA,
            pltpu.SemaphoreType.DMA,
        ],
    )
    def k(table_hbm, idx_hbm, out_hbm,
          idx_v, zbuf, b0_, b1_, b2_, b3_, b4_,
          isem, zsem, g0_, g1_, g2_, g3_, g4_,
          w0_, w1_, w2_, w3_, w4_):
        wid = lax.axis_index("c") * NS + lax.axis_index("s")
        b0 = wid * BW
        bufs = (b0_, b1_, b2_, b3_, b4_)
        gsems = (g0_, g1_, g2_, g3_, g4_)
        wsems = (w0_, w1_, w2_, w3_, w4_)

        idx_cp = [
            pltpu.async_copy(idx_hbm.at[:, pl.ds(b0, BW)], idx_v, isem)
        ]

        zeros16 = jnp.zeros((16,), jnp.float32)

        def zfill(r, carry):
            for cc in range(N_WORD // 16):
                zbuf[r, pl.ds(cc * 16, 16)] = zeros16
            return carry

        lax.fori_loop(0, BW, zfill, 0)

        zw0 = pltpu.async_copy(zbuf, out_hbm.at[0, pl.ds(b0, BW)], zsem)
        zw1 = pltpu.async_copy(zbuf, out_hbm.at[LP - 1, pl.ds(b0, BW)], zsem)

        for cp in idx_cp:
            cp.wait()

        def fire_gather(i, buf, sem):
            for h in range(2):
                pltpu.async_copy(
                    table_hbm.at[idx_v.at[i, pl.ds(h * (BW // 2), BW // 2)]],
                    buf.at[pl.ds(h * (BW // 2), BW // 2)],
                    sem,
                )

        def wait_gather(i, buf, sem):
            for h in range(2):
                pltpu.make_async_copy(
                    table_hbm.at[idx_v.at[i, pl.ds(h * (BW // 2), BW // 2)]],
                    buf.at[pl.ds(h * (BW // 2), BW // 2)],
                    sem,
                ).wait()

        def fire_write(i, buf, sem):
            pltpu.async_copy(buf, out_hbm.at[i + 1, pl.ds(b0, BW)], sem)

        def wait_write(i, buf, sem):
            pltpu.make_async_copy(
                buf, out_hbm.at[i + 1, pl.ds(b0, BW)], sem
            ).wait()

        for j in range(NBUF):
            fire_gather(j, bufs[j], gsems[j])

        def body(p, carry):
            i = p * NBUF
            for j in range(NBUF):
                wait_gather(i + j, bufs[j], gsems[j])
                fire_write(i + j, bufs[j], wsems[j])
            for j in range(NBUF):
                wait_write(i + j, bufs[j], wsems[j])
                fire_gather(i + NBUF + j, bufs[j], gsems[j])
            return carry

        lax.fori_loop(0, L // NBUF - 2, body, 0)

        i = L - 2 * NBUF
        for j in range(NBUF):
            wait_gather(i + j, bufs[j], gsems[j])
            fire_write(i + j, bufs[j], wsems[j])
        for j in range(NBUF):
            wait_write(i + j, bufs[j], wsems[j])
            fire_gather(i + NBUF + j, bufs[j], gsems[j])
        i = L - NBUF
        for j in range(NBUF):
            wait_gather(i + j, bufs[j], gsems[j])
            fire_write(i + j, bufs[j], wsems[j])
        for j in range(NBUF):
            wait_write(i + j, bufs[j], wsems[j])
        zw0.wait()
        zw1.wait()

    return k(table, idx_t)


def kernel(table, indices):
    out = _sc_embed(table, indices.astype(jnp.int32).T)
    return out.transpose(1, 0, 2)


# speedup vs baseline: 1.0086x; 1.0086x over previous
import functools

import jax
import jax.numpy as jnp
from jax import lax
from jax.experimental import pallas as pl
from jax.experimental.pallas import tpu as pltpu
from jax.experimental.pallas import tpu_sc as plsc

N_WORD = 128
B = 4096
L = 50
LP = L + 2

NC = 2
NS = 16
NW = NC * NS
BW = B // NW
NBUF = 5


def _sc_embed(table, idx_t):
    mesh = plsc.VectorSubcoreMesh(core_axis_name="c", subcore_axis_name="s")

    @functools.partial(
        pl.kernel,
        mesh=mesh,
        out_type=jax.ShapeDtypeStruct((LP, B, N_WORD), jnp.float32),
        scratch_types=[
            pltpu.VMEM((L, BW), jnp.int32),
            pltpu.VMEM((BW, N_WORD), jnp.float32),
            pltpu.VMEM((BW, N_WORD), jnp.float32),
            pltpu.VMEM((BW, N_WORD), jnp.float32),
            pltpu.VMEM((BW, N_WORD), jnp.float32),
            pltpu.VMEM((BW, N_WORD), jnp.float32),
            pltpu.VMEM((BW, N_WORD), jnp.float32),
            pltpu.SemaphoreType.DMA,
            pltpu.SemaphoreType.DMA,
            pltpu.SemaphoreType.DMA,
            pltpu.SemaphoreType.DMA,
            pltpu.SemaphoreType.DMA,
            pltpu.SemaphoreType.DMA,
            pltpu.SemaphoreType.DMA,
            pltpu.SemaphoreType.DMA,
            pltpu.SemaphoreType.DMA,
            pltpu.SemaphoreType.DMA,
            pltpu.SemaphoreType.DMA,
            pltpu.SemaphoreType.DMA,
        ],
    )
    def k(table_hbm, idx_hbm, out_hbm,
          idx_v, zbuf, b0_, b1_, b2_, b3_, b4_,
          isem, zsem, g0_, g1_, g2_, g3_, g4_,
          w0_, w1_, w2_, w3_, w4_):
        wid = lax.axis_index("c") * NS + lax.axis_index("s")
        b0 = wid * BW
        bufs = (b0_, b1_, b2_, b3_, b4_)
        gsems = (g0_, g1_, g2_, g3_, g4_)
        wsems = (w0_, w1_, w2_, w3_, w4_)

        idx_cp = [
            pltpu.async_copy(idx_hbm.at[:, pl.ds(b0, BW)], idx_v, isem)
        ]

        zeros16 = jnp.zeros((16,), jnp.float32)

        def zfill(r, carry):
            for cc in range(N_WORD // 16):
                zbuf[r, pl.ds(cc * 16, 16)] = zeros16
            return carry

        lax.fori_loop(0, BW, zfill, 0)

        zw0 = pltpu.async_copy(zbuf, out_hbm.at[0, pl.ds(b0, BW)], zsem)
        zw1 = pltpu.async_copy(zbuf, out_hbm.at[LP - 1, pl.ds(b0, BW)], zsem)

        for cp in idx_cp:
            cp.wait()

        def fire_gather(i, buf, sem):
            pltpu.async_copy(table_hbm.at[idx_v.at[i]], buf, sem)

        def wait_gather(i, buf, sem):
            pltpu.make_async_copy(table_hbm.at[idx_v.at[i]], buf, sem).wait()

        def fire_write(i, buf, sem):
            pltpu.async_copy(buf, out_hbm.at[i + 1, pl.ds(b0, BW)], sem)

        def wait_write(i, buf, sem):
            pltpu.make_async_copy(
                buf, out_hbm.at[i + 1, pl.ds(b0, BW)], sem
            ).wait()

        for j in range(NBUF):
            fire_gather(j, bufs[j], gsems[j])

        def body(p, carry):
            i = p * NBUF
            for j in range(NBUF):
                wait_gather(i + j, bufs[j], gsems[j])
                fire_write(i + j, bufs[j], wsems[j])
            for j in range(NBUF):
                wait_write(i + j, bufs[j], wsems[j])
                fire_gather(i + NBUF + j, bufs[j], gsems[j])
            return carry

        lax.fori_loop(0, L // NBUF - 2, body, 0)

        i = L - 2 * NBUF
        for j in range(NBUF):
            wait_gather(i + j, bufs[j], gsems[j])
            fire_write(i + j, bufs[j], wsems[j])
        for j in range(NBUF):
            wait_write(i + j, bufs[j], wsems[j])
            fire_gather(i + NBUF + j, bufs[j], gsems[j])
        i = L - NBUF
        for j in range(NBUF):
            wait_gather(i + j, bufs[j], gsems[j])
            fire_write(i + j, bufs[j], wsems[j])
        for j in range(NBUF):
            wait_write(i + j, bufs[j], wsems[j])
        zw0.wait()
        zw1.wait()

    return k(table, idx_t)


def kernel(table, indices):
    out = _sc_embed(table, indices.astype(jnp.int32).T)
    return out.transpose(1, 0, 2)
